# baseline (speedup 1.0000x reference)
import functools

import jax
import jax.numpy as jnp
from jax import lax
from jax.experimental import pallas as pl
from jax.experimental.pallas import tpu as pltpu
from jax.experimental.pallas import tpu_sc as plsc

DIM = 32
NC = 2
NS = 16
NW = NC * NS
K = 128
CH_ROWS = 10
C = CH_ROWS * K


def _make_gather(total, vocab, dim):
    b_per_w = total // NW
    n_idx_rows = b_per_w // K
    n_chunks = n_idx_rows // CH_ROWS

    mesh = plsc.VectorSubcoreMesh(core_axis_name="c", subcore_axis_name="s")

    @functools.partial(
        pl.kernel,
        mesh=mesh,
        out_type=jax.ShapeDtypeStruct((total, dim), jnp.float32),
        compiler_params=pltpu.CompilerParams(use_tc_tiling_on_sc=False),
        scratch_types=[
            pltpu.VMEM((n_idx_rows, K), jnp.int32),
            pltpu.VMEM((C, dim), jnp.float32),
            pltpu.VMEM((C, dim), jnp.float32),
            pltpu.SemaphoreType.DMA,
            pltpu.SemaphoreType.DMA,
            pltpu.SemaphoreType.DMA,
            pltpu.SemaphoreType.DMA,
        ],
    )
    def gather_kernel(idx_hbm, table_hbm, out_hbm, idx_v, buf0, buf1,
                      gsem0, gsem1, osem0, osem1):
        wid = lax.axis_index("s") * NC + lax.axis_index("c")
        base = wid * b_per_w
        pltpu.sync_copy(idx_hbm.at[wid], idx_v)

        bufs = (buf0, buf1)
        gsems = (gsem0, gsem1)
        osems = (osem0, osem1)

        gathers = [None] * n_chunks
        writes = [None] * n_chunks

        def fire(c):
            buf, sem = bufs[c % 2], gsems[c % 2]
            cps = []
            for j in range(CH_ROWS):
                cps.append(
                    pltpu.async_copy(
                        table_hbm.at[idx_v.at[c * CH_ROWS + j]],
                        buf.at[pl.ds(j * K, K)],
                        sem,
                    )
                )
            gathers[c] = cps

        fire(0)
        for c in range(n_chunks):
            if c + 1 < n_chunks:
                nb = (c + 1) % 2
                if writes[nb] is not None:
                    writes[nb].wait()
                    writes[nb] = None
                fire(c + 1)
            for cp in gathers[c]:
                cp.wait()
            writes[c % 2] = pltpu.async_copy(
                bufs[c % 2],
                out_hbm.at[pl.ds(base + c * C, C)],
                osems[c % 2],
            )
        for w in writes:
            if w is not None:
                w.wait()

    return gather_kernel


def kernel(indices, row_splits, table):
    total = indices.shape[0]
    vocab, dim = table.shape
    idx3 = indices.reshape(NW, total // NW // K, K)
    return _make_gather(total, vocab, dim)(idx3, table)
